# baseline (speedup 1.0000x reference)
import jax
import jax.numpy as jnp
from jax.experimental import pallas as pl
from jax.experimental.pallas import tpu as pltpu


_ROWS = 8192
_COLS = 1024
_NCORE = 2
_ROWS_PER_CORE = _ROWS // _NCORE
_NB = 4
_CHUNK = 256
_N_CHUNKS = _ROWS_PER_CORE // _CHUNK
_LAG = 2


def _copy_kernel(x_hbm, o_hbm, bufs, lsem, ssem):
    tid = pl.program_id(0)
    base = tid * _ROWS_PER_CORE

    def load(i):
        return pltpu.make_async_copy(
            x_hbm.at[pl.ds(base + i * _CHUNK, _CHUNK), :],
            bufs.at[i % _NB],
            lsem.at[i % _NB],
        )

    def store(i):
        return pltpu.make_async_copy(
            bufs.at[i % _NB],
            o_hbm.at[pl.ds(base + i * _CHUNK, _CHUNK), :],
            ssem.at[i % _NB],
        )

    for b in range(_NB):
        load(b).start()
    for i in range(_N_CHUNKS):
        j = i - _LAG
        if j >= 0:
            store(j).wait()
            if j + _NB < _N_CHUNKS:
                load(j + _NB).start()
        load(i).wait()
        store(i).start()
    for i in range(_N_CHUNKS - _LAG, _N_CHUNKS):
        store(i).wait()


def kernel(x):
    gathered = pl.pallas_call(
        _copy_kernel,
        grid=(_NCORE,),
        in_specs=[pl.BlockSpec(memory_space=pl.ANY)],
        out_specs=pl.BlockSpec(memory_space=pl.ANY),
        out_shape=jax.ShapeDtypeStruct((_ROWS, _COLS), x.dtype),
        scratch_shapes=[
            pltpu.VMEM((_NB, _CHUNK, _COLS), jnp.float32),
            pltpu.SemaphoreType.DMA((_NB,)),
            pltpu.SemaphoreType.DMA((_NB,)),
        ],
        compiler_params=pltpu.CompilerParams(
            dimension_semantics=("parallel",),
        ),
    )(x)
    sizes = jnp.array([_ROWS], dtype=jnp.int32)
    return (gathered, sizes)


# speedup vs baseline: 1.2337x; 1.2337x over previous
import jax
import jax.numpy as jnp
from jax.experimental import pallas as pl
from jax.experimental.pallas import tpu as pltpu


_ROWS = 8192
_COLS = 1024
_BLOCK_ROWS = 2048


def _copy_kernel(x_ref, o_ref):
    o_ref[...] = x_ref[...]


def kernel(x):
    n_blocks = _ROWS // _BLOCK_ROWS
    gathered = pl.pallas_call(
        _copy_kernel,
        grid=(n_blocks,),
        in_specs=[pl.BlockSpec((_BLOCK_ROWS, _COLS), lambda i: (i, 0))],
        out_specs=pl.BlockSpec((_BLOCK_ROWS, _COLS), lambda i: (i, 0)),
        out_shape=jax.ShapeDtypeStruct((_ROWS, _COLS), x.dtype),
        compiler_params=pltpu.CompilerParams(
            dimension_semantics=("parallel",),
        ),
    )(x)
    sizes = jnp.array([_ROWS], dtype=jnp.int32)
    return (gathered, sizes)
